# baseline (speedup 1.0000x reference)
import functools

import jax
import jax.numpy as jnp
from jax import lax
from jax.experimental import pallas as pl
from jax.experimental.pallas import tpu as pltpu
from jax.experimental.pallas import tpu_sc as plsc

B, C, H, W = 4, 192, 224, 224
HEADS = 4
N = B * 4
ROWS = B * C
CB = 32
NW = 32
RPT = ROWS // NW
LH = W // 16


def _means_body(x_ref, r_ref, o_ref):
    inv = 1.0 / (H * W)
    o_ref[0, 0:1, :] = jnp.sum(x_ref[...], axis=(2, 3)) * inv
    o_ref[0, 1:2, :] = jnp.sum(r_ref[...], axis=(2, 3)) * inv


def _means(x, r):
    grid = ROWS // CB
    nc = C // CB
    bs = pl.BlockSpec((1, CB, H, W), lambda i: (i // nc, i % nc, 0, 0))
    out = pl.pallas_call(
        _means_body,
        grid=(grid,),
        in_specs=[bs] * 2,
        out_specs=pl.BlockSpec((1, 2, CB), lambda i: (i, 0, 0)),
        out_shape=jax.ShapeDtypeStruct((grid, 2, CB), jnp.float32),
    )(x, r)
    return out.transpose(1, 0, 2).reshape(2, B, C)


def _sc_mean_body(dep_hbm, out_hbm, buf, acc, sem0, sem1):
    wid = lax.axis_index("s") * 2 + lax.axis_index("c")
    base = wid * RPT
    sems = (sem0, sem1)

    def start(j):
        row = base + j
        return pltpu.async_copy(dep_hbm.at[row // C, row % C],
                                buf.at[j % 2], sems[j % 2])

    pending = [None, None]
    pending[0] = start(0)
    inv = 1.0 / (H * W)
    for j in range(RPT):
        cur = j % 2
        if j + 1 < RPT:
            pending[(j + 1) % 2] = start(j + 1)
        pending[cur].wait()

        def body(rr, accs):
            return tuple(a + buf[cur, rr, pl.ds(t * 16, 16)]
                         for t, a in enumerate(accs))

        accs = lax.fori_loop(
            0, H, body,
            tuple(jnp.zeros((16,), jnp.float32) for _ in range(LH)))
        tot = accs[0]
        for t in range(1, LH):
            tot = tot + accs[t]
        acc[j] = tot * inv
    pltpu.sync_copy(acc, out_hbm.at[base // C, pl.ds(base % C, RPT)])


def _sc_mean(dep):
    mesh = plsc.VectorSubcoreMesh(core_axis_name="c", subcore_axis_name="s")
    f = pl.kernel(
        _sc_mean_body,
        out_type=jax.ShapeDtypeStruct((B, C, 16), jnp.float32),
        mesh=mesh,
        scratch_types=[pltpu.VMEM((2, H, W), jnp.float32),
                       pltpu.VMEM((RPT, 16), jnp.float32),
                       pltpu.SemaphoreType.DMA,
                       pltpu.SemaphoreType.DMA],
    )
    return f(dep)


def _adj_mask():
    r = lax.broadcasted_iota(jnp.int32, (N, N), 0)
    c = lax.broadcasted_iota(jnp.int32, (N, N), 1)
    same = (r // 4) == (c // 4)
    nr, nc = r % 4, c % 4
    adj = (nc >= 1) | ((nr == 0) & (nc == 0))
    return same & adj


def _gat_layer(g, Wm, a_s, a_d, bb, mask, maskf):
    h = jnp.dot(g, Wm, preferred_element_type=jnp.float32)
    acc = jnp.zeros((N, C), jnp.float32)
    for hd in range(HEADS):
        hh = h[:, hd * C:(hd + 1) * C]
        a_s_h = a_s[hd:hd + 1, :]
        a_d_h = a_d[hd:hd + 1, :]
        al_s = lax.dot_general(a_s_h, hh, (((1,), (1,)), ((), ())),
                               preferred_element_type=jnp.float32)
        al_d = lax.dot_general(hh, a_d_h, (((1,), (1,)), ((), ())),
                               preferred_element_type=jnp.float32)
        e = al_d + al_s
        e = jnp.where(e > 0, e, 0.2 * e)
        e = jnp.where(mask, e, -1e30)
        m = jnp.max(e, axis=1, keepdims=True)
        ex = jnp.exp(e - m) * maskf
        ssum = jnp.sum(ex, axis=1, keepdims=True) + 1e-16
        alpha = ex / ssum
        acc = acc + jnp.dot(alpha, hh, preferred_element_type=jnp.float32)
    return acc * (1.0 / HEADS) + bb


def _ln(x, g, b):
    mu = jnp.mean(x, axis=-1, keepdims=True)
    xc = x - mu
    var = jnp.mean(xc * xc, axis=-1, keepdims=True)
    return xc * lax.rsqrt(var + 1e-5) * g + b


def _gnn_body(tok_ref, mean_ref, depp_ref, W0_ref, as0_ref, ad0_ref, b0_ref,
              g0_ref, be0_ref, W1_ref, as1_ref, ad1_ref, b1_ref, g1_ref,
              be1_ref, o_ref):
    t = jnp.broadcast_to(tok_ref[...], (B, C))
    fu = mean_ref[0]
    x1 = mean_ref[1]
    x2 = jnp.sum(depp_ref[...], axis=2)
    feats = jnp.stack([t, fu, x1, x2], axis=1).reshape(N, C)

    mask = _adj_mask()
    maskf = mask.astype(jnp.float32)

    g = feats
    for (Wr, ar_s, ar_d, br, lgr, lbr) in (
            (W0_ref, as0_ref, ad0_ref, b0_ref, g0_ref, be0_ref),
            (W1_ref, as1_ref, ad1_ref, b1_ref, g1_ref, be1_ref)):
        g = _gat_layer(g, Wr[...], ar_s[...], ar_d[...], br[...], mask,
                       maskf) + g
        g = _ln(g, lgr[...], lbr[...])
        g = jnp.maximum(g, 0.0)

    rr = lax.broadcasted_iota(jnp.int32, (B, N), 0)
    cc = lax.broadcasted_iota(jnp.int32, (B, N), 1)
    sel = (cc == rr * 4).astype(jnp.float32)
    gtok = jnp.dot(sel, g, preferred_element_type=jnp.float32)
    o_ref[...] = 1.0 + jax.nn.sigmoid(gtok)


def _gnn(tok, means, depp, W0, as0, ad0, b0, g0, be0, W1, as1, ad1, b1, g1,
         be1):
    full = lambda s: pl.BlockSpec(s, lambda: (0,) * len(s))
    return pl.pallas_call(
        _gnn_body,
        in_specs=[full((1, C)), full((2, B, C)), full((B, C, 16)),
                  full((C, HEADS * C)),
                  full((HEADS, C)), full((HEADS, C)), full((1, C)),
                  full((1, C)), full((1, C)), full((C, HEADS * C)),
                  full((HEADS, C)), full((HEADS, C)), full((1, C)),
                  full((1, C)), full((1, C))],
        out_specs=full((B, C)),
        out_shape=jax.ShapeDtypeStruct((B, C), jnp.float32),
    )(tok, means, depp, W0, as0, ad0, b0, g0, be0, W1, as1, ad1, b1, g1, be1)


def _scale_body(x_ref, s_ref, o_ref):
    i = pl.program_id(0)
    for k in range(CB):
        o_ref[0, k] = x_ref[0, k] * s_ref[i * CB + k]


def _scale(x, s):
    grid = ROWS // CB
    nc = C // CB
    bs = pl.BlockSpec((1, CB, H, W), lambda i: (i // nc, i % nc, 0, 0))
    return pl.pallas_call(
        _scale_body,
        grid=(grid,),
        in_specs=[bs, pl.BlockSpec(memory_space=pltpu.SMEM)],
        out_specs=bs,
        out_shape=jax.ShapeDtypeStruct((B, C, H, W), jnp.float32),
    )(x, s)


def kernel(x_ful, rgb, dep, tok, W0, a_src0, a_dst0, b0, g0, be0,
           W1, a_src1, a_dst1, b1, g1, be1):
    depp = _sc_mean(dep)
    means = _means(x_ful, rgb)

    scale = _gnn(tok, means, depp,
                 W0, a_src0.reshape(HEADS, C), a_dst0.reshape(HEADS, C),
                 b0.reshape(1, C), g0.reshape(1, C), be0.reshape(1, C),
                 W1, a_src1.reshape(HEADS, C), a_dst1.reshape(HEADS, C),
                 b1.reshape(1, C), g1.reshape(1, C), be1.reshape(1, C))

    return _scale(x_ful, scale.reshape(ROWS))


# speedup vs baseline: 1.0948x; 1.0948x over previous
import functools

import jax
import jax.numpy as jnp
from jax import lax
from jax.experimental import pallas as pl
from jax.experimental.pallas import tpu as pltpu

B, C, H, W = 4, 192, 224, 224
HEADS = 4
N = B * 4
ROWS = B * C
CB = 32
NC = C // CB


def _means_body(x_ref, r_ref, d_ref, o_ref):
    inv = 1.0 / (H * W)
    o_ref[0, 0:1, :] = jnp.sum(x_ref[...], axis=(2, 3)) * inv
    o_ref[0, 1:2, :] = jnp.sum(r_ref[...], axis=(2, 3)) * inv
    o_ref[0, 2:3, :] = jnp.sum(d_ref[...], axis=(2, 3)) * inv


def _means(x, r, d):
    grid = ROWS // CB
    bs = pl.BlockSpec((1, CB, H, W), lambda i: (i // NC, i % NC, 0, 0))
    return pl.pallas_call(
        _means_body,
        grid=(grid,),
        in_specs=[bs] * 3,
        out_specs=pl.BlockSpec((1, 3, CB), lambda i: (i, 0, 0)),
        out_shape=jax.ShapeDtypeStruct((grid, 3, CB), jnp.float32),
    )(x, r, d)


def _adj_mask():
    r = lax.broadcasted_iota(jnp.int32, (N, N), 0)
    c = lax.broadcasted_iota(jnp.int32, (N, N), 1)
    same = (r // 4) == (c // 4)
    nr, nc = r % 4, c % 4
    adj = (nc >= 1) | ((nr == 0) & (nc == 0))
    return same & adj


def _gat_layer(g, Wm, a_s, a_d, bb, mask, maskf):
    h = jnp.dot(g, Wm, preferred_element_type=jnp.float32)
    acc = jnp.zeros((N, C), jnp.float32)
    for hd in range(HEADS):
        hh = h[:, hd * C:(hd + 1) * C]
        a_s_h = a_s[hd:hd + 1, :]
        a_d_h = a_d[hd:hd + 1, :]
        al_s = lax.dot_general(a_s_h, hh, (((1,), (1,)), ((), ())),
                               preferred_element_type=jnp.float32)
        al_d = lax.dot_general(hh, a_d_h, (((1,), (1,)), ((), ())),
                               preferred_element_type=jnp.float32)
        e = al_d + al_s
        e = jnp.where(e > 0, e, 0.2 * e)
        e = jnp.where(mask, e, -1e30)
        m = jnp.max(e, axis=1, keepdims=True)
        ex = jnp.exp(e - m) * maskf
        ssum = jnp.sum(ex, axis=1, keepdims=True) + 1e-16
        alpha = ex / ssum
        acc = acc + jnp.dot(alpha, hh, preferred_element_type=jnp.float32)
    return acc * (1.0 / HEADS) + bb


def _ln(x, g, b):
    mu = jnp.mean(x, axis=-1, keepdims=True)
    xc = x - mu
    var = jnp.mean(xc * xc, axis=-1, keepdims=True)
    return xc * lax.rsqrt(var + 1e-5) * g + b


def _gnn_body(tok_ref, mean_ref, W0_ref, as0_ref, ad0_ref, b0_ref, g0_ref,
              be0_ref, W1_ref, as1_ref, ad1_ref, b1_ref, g1_ref, be1_ref,
              o_ref):
    t = jnp.broadcast_to(tok_ref[...], (B, C))
    fu = mean_ref[0]
    x1 = mean_ref[1]
    x2 = mean_ref[2]
    feats = jnp.stack([t, fu, x1, x2], axis=1).reshape(N, C)

    mask = _adj_mask()
    maskf = mask.astype(jnp.float32)

    g = feats
    for (Wr, ar_s, ar_d, br, lgr, lbr) in (
            (W0_ref, as0_ref, ad0_ref, b0_ref, g0_ref, be0_ref),
            (W1_ref, as1_ref, ad1_ref, b1_ref, g1_ref, be1_ref)):
        g = _gat_layer(g, Wr[...], ar_s[...], ar_d[...], br[...], mask,
                       maskf) + g
        g = _ln(g, lgr[...], lbr[...])
        g = jnp.maximum(g, 0.0)

    rr = lax.broadcasted_iota(jnp.int32, (B, N), 0)
    cc = lax.broadcasted_iota(jnp.int32, (B, N), 1)
    sel = (cc == rr * 4).astype(jnp.float32)
    gtok = jnp.dot(sel, g, preferred_element_type=jnp.float32)
    o_ref[...] = 1.0 + jax.nn.sigmoid(gtok)


def _gnn(tok, means_raw, W0, as0, ad0, b0, g0, be0, W1, as1, ad1, b1, g1,
         be1):
    full = lambda s: pl.BlockSpec(s, lambda: (0,) * len(s))
    return pl.pallas_call(
        _gnn_body,
        in_specs=[full((1, C)), full((3, B, C)),
                  full((C, HEADS * C)),
                  full((HEADS, C)), full((HEADS, C)), full((1, C)),
                  full((1, C)), full((1, C)), full((C, HEADS * C)),
                  full((HEADS, C)), full((HEADS, C)), full((1, C)),
                  full((1, C)), full((1, C))],
        out_specs=full((B, C)),
        out_shape=jax.ShapeDtypeStruct((B, C), jnp.float32),
    )(tok, means_raw, W0, as0, ad0, b0, g0, be0, W1, as1, ad1, b1, g1, be1)


def _scale_body(x_ref, s_ref, o_ref):
    i = pl.program_id(0)
    b = i // NC
    c0 = (i % NC) * CB
    for k in range(CB):
        o_ref[0, k] = x_ref[0, k] * s_ref[b, c0 + k]


def _scale(x, s):
    grid = ROWS // CB
    bs = pl.BlockSpec((1, CB, H, W), lambda i: (i // NC, i % NC, 0, 0))
    return pl.pallas_call(
        _scale_body,
        grid=(grid,),
        in_specs=[bs, pl.BlockSpec(memory_space=pltpu.SMEM)],
        out_specs=bs,
        out_shape=jax.ShapeDtypeStruct((B, C, H, W), jnp.float32),
    )(x, s)


def kernel(x_ful, rgb, dep, tok, W0, a_src0, a_dst0, b0, g0, be0,
           W1, a_src1, a_dst1, b1, g1, be1):
    means = _means(x_ful, rgb, dep).transpose(1, 0, 2).reshape(3, B, C)

    scale = _gnn(tok, means,
                 W0, a_src0.reshape(HEADS, C), a_dst0.reshape(HEADS, C),
                 b0.reshape(1, C), g0.reshape(1, C), be0.reshape(1, C),
                 W1, a_src1.reshape(HEADS, C), a_dst1.reshape(HEADS, C),
                 b1.reshape(1, C), g1.reshape(1, C), be1.reshape(1, C))

    return _scale(x_ful, scale)


# speedup vs baseline: 1.1075x; 1.0116x over previous
import functools

import jax
import jax.numpy as jnp
from jax import lax
from jax.experimental import pallas as pl
from jax.experimental.pallas import tpu as pltpu

B, C, H, W = 4, 192, 224, 224
HEADS = 4
N = B * 4
ROWS = B * C
CB = 32
NC = C // CB


def _means_gnn_body(x_ref, r_ref, d_ref, tok_ref, W0_ref, as0_ref, ad0_ref,
                    b0_ref, g0_ref, be0_ref, W1_ref, as1_ref, ad1_ref,
                    b1_ref, g1_ref, be1_ref, o_ref, acc_ref):
    grid = ROWS // CB
    i = pl.program_id(0)
    inv = 1.0 / (H * W)
    acc_ref[pl.ds(i, 1), 0:1, :] = jnp.sum(
        x_ref[...], axis=(2, 3)).reshape(1, 1, CB) * inv
    acc_ref[pl.ds(i, 1), 1:2, :] = jnp.sum(
        r_ref[...], axis=(2, 3)).reshape(1, 1, CB) * inv
    acc_ref[pl.ds(i, 1), 2:3, :] = jnp.sum(
        d_ref[...], axis=(2, 3)).reshape(1, 1, CB) * inv

    @pl.when(i == grid - 1)
    def _gnn_step():
        rows = []
        for b in range(B):
            rows.append(tok_ref[...])
            for t in range(3):
                rows.append(jnp.concatenate(
                    [acc_ref[b * NC + j, t:t + 1, :] for j in range(NC)],
                    axis=1))
        feats = jnp.concatenate(rows, axis=0)
        _gnn_compute(feats, W0_ref, as0_ref, ad0_ref, b0_ref,
                     g0_ref, be0_ref, W1_ref, as1_ref, ad1_ref, b1_ref,
                     g1_ref, be1_ref, o_ref)


def _means_gnn(x, r, d, tok, W0, as0, ad0, b0, g0, be0, W1, as1, ad1, b1,
               g1, be1):
    grid = ROWS // CB
    bs = pl.BlockSpec((1, CB, H, W), lambda i: (i // NC, i % NC, 0, 0))
    full = lambda s: pl.BlockSpec(s, lambda i: (0,) * len(s))
    return pl.pallas_call(
        _means_gnn_body,
        grid=(grid,),
        in_specs=[bs, bs, bs,
                  full((1, C)), full((C, HEADS * C)),
                  full((HEADS, C)), full((HEADS, C)), full((1, C)),
                  full((1, C)), full((1, C)), full((C, HEADS * C)),
                  full((HEADS, C)), full((HEADS, C)), full((1, C)),
                  full((1, C)), full((1, C))],
        out_specs=full((B, C)),
        out_shape=jax.ShapeDtypeStruct((B, C), jnp.float32),
        scratch_shapes=[pltpu.VMEM((grid, 3, CB), jnp.float32)],
    )(x, r, d, tok, W0, as0, ad0, b0, g0, be0, W1, as1, ad1, b1, g1, be1)


def _adj_mask():
    r = lax.broadcasted_iota(jnp.int32, (N, N), 0)
    c = lax.broadcasted_iota(jnp.int32, (N, N), 1)
    same = (r // 4) == (c // 4)
    nr, nc = r % 4, c % 4
    adj = (nc >= 1) | ((nr == 0) & (nc == 0))
    return same & adj


def _gat_layer(g, Wm, a_s, a_d, bb, mask, maskf):
    h = jnp.dot(g, Wm, preferred_element_type=jnp.float32)
    acc = jnp.zeros((N, C), jnp.float32)
    for hd in range(HEADS):
        hh = h[:, hd * C:(hd + 1) * C]
        a_s_h = a_s[hd:hd + 1, :]
        a_d_h = a_d[hd:hd + 1, :]
        al_s = lax.dot_general(a_s_h, hh, (((1,), (1,)), ((), ())),
                               preferred_element_type=jnp.float32)
        al_d = lax.dot_general(hh, a_d_h, (((1,), (1,)), ((), ())),
                               preferred_element_type=jnp.float32)
        e = al_d + al_s
        e = jnp.where(e > 0, e, 0.2 * e)
        e = jnp.where(mask, e, -1e30)
        m = jnp.max(e, axis=1, keepdims=True)
        ex = jnp.exp(e - m) * maskf
        ssum = jnp.sum(ex, axis=1, keepdims=True) + 1e-16
        alpha = ex / ssum
        acc = acc + jnp.dot(alpha, hh, preferred_element_type=jnp.float32)
    return acc * (1.0 / HEADS) + bb


def _ln(x, g, b):
    mu = jnp.mean(x, axis=-1, keepdims=True)
    xc = x - mu
    var = jnp.mean(xc * xc, axis=-1, keepdims=True)
    return xc * lax.rsqrt(var + 1e-5) * g + b


def _gnn_compute(feats, W0_ref, as0_ref, ad0_ref, b0_ref,
                 g0_ref, be0_ref, W1_ref, as1_ref, ad1_ref, b1_ref, g1_ref,
                 be1_ref, o_ref):
    mask = _adj_mask()
    maskf = mask.astype(jnp.float32)

    g = feats
    for (Wr, ar_s, ar_d, br, lgr, lbr) in (
            (W0_ref, as0_ref, ad0_ref, b0_ref, g0_ref, be0_ref),
            (W1_ref, as1_ref, ad1_ref, b1_ref, g1_ref, be1_ref)):
        g = _gat_layer(g, Wr[...], ar_s[...], ar_d[...], br[...], mask,
                       maskf) + g
        g = _ln(g, lgr[...], lbr[...])
        g = jnp.maximum(g, 0.0)

    rr = lax.broadcasted_iota(jnp.int32, (B, N), 0)
    cc = lax.broadcasted_iota(jnp.int32, (B, N), 1)
    sel = (cc == rr * 4).astype(jnp.float32)
    gtok = jnp.dot(sel, g, preferred_element_type=jnp.float32)
    o_ref[...] = 1.0 + jax.nn.sigmoid(gtok)


def _scale_body(x_ref, s_ref, o_ref):
    i = pl.program_id(0)
    b = i // NC
    c0 = (i % NC) * CB
    for k in range(CB):
        o_ref[0, k] = x_ref[0, k] * s_ref[b, c0 + k]


def _scale(x, s):
    grid = ROWS // CB
    bs = pl.BlockSpec((1, CB, H, W), lambda i: (i // NC, i % NC, 0, 0))
    return pl.pallas_call(
        _scale_body,
        grid=(grid,),
        in_specs=[bs, pl.BlockSpec(memory_space=pltpu.SMEM)],
        out_specs=bs,
        out_shape=jax.ShapeDtypeStruct((B, C, H, W), jnp.float32),
    )(x, s)


def kernel(x_ful, rgb, dep, tok, W0, a_src0, a_dst0, b0, g0, be0,
           W1, a_src1, a_dst1, b1, g1, be1):
    scale = _means_gnn(
        x_ful, rgb, dep, tok,
        W0, a_src0.reshape(HEADS, C), a_dst0.reshape(HEADS, C),
        b0.reshape(1, C), g0.reshape(1, C), be0.reshape(1, C),
        W1, a_src1.reshape(HEADS, C), a_dst1.reshape(HEADS, C),
        b1.reshape(1, C), g1.reshape(1, C), be1.reshape(1, C))

    return _scale(x_ful, scale)


# speedup vs baseline: 1.1111x; 1.0032x over previous
import functools

import jax
import jax.numpy as jnp
from jax import lax
from jax.experimental import pallas as pl
from jax.experimental.pallas import tpu as pltpu

B, C, H, W = 4, 192, 224, 224
HEADS = 4
N = B * 4
ROWS = B * C
CB = 32
NC = C // CB
CB2 = 64
NC2 = C // CB2


def _means_gnn_body(x_ref, r_ref, d_ref, tok_ref, W0_ref, as0_ref, ad0_ref,
                    b0_ref, g0_ref, be0_ref, W1_ref, as1_ref, ad1_ref,
                    b1_ref, g1_ref, be1_ref, o_ref, acc_ref):
    grid = ROWS // CB
    i = pl.program_id(0)
    inv = 1.0 / (H * W)
    acc_ref[pl.ds(i, 1), 0:1, :] = jnp.sum(
        x_ref[...], axis=(2, 3)).reshape(1, 1, CB) * inv
    acc_ref[pl.ds(i, 1), 1:2, :] = jnp.sum(
        r_ref[...], axis=(2, 3)).reshape(1, 1, CB) * inv
    acc_ref[pl.ds(i, 1), 2:3, :] = jnp.sum(
        d_ref[...], axis=(2, 3)).reshape(1, 1, CB) * inv

    @pl.when(i == grid - 1)
    def _gnn_step():
        rows = []
        for b in range(B):
            rows.append(tok_ref[...])
            for t in range(3):
                rows.append(jnp.concatenate(
                    [acc_ref[b * NC + j, t:t + 1, :] for j in range(NC)],
                    axis=1))
        feats = jnp.concatenate(rows, axis=0)
        _gnn_compute(feats, W0_ref, as0_ref, ad0_ref, b0_ref,
                     g0_ref, be0_ref, W1_ref, as1_ref, ad1_ref, b1_ref,
                     g1_ref, be1_ref, o_ref)


def _means_gnn(x, r, d, tok, W0, as0, ad0, b0, g0, be0, W1, as1, ad1, b1,
               g1, be1):
    grid = ROWS // CB
    bs = pl.BlockSpec((1, CB, H, W), lambda i: (i // NC, i % NC, 0, 0))
    full = lambda s: pl.BlockSpec(s, lambda i: (0,) * len(s))
    return pl.pallas_call(
        _means_gnn_body,
        grid=(grid,),
        in_specs=[bs, bs, bs,
                  full((1, C)), full((C, HEADS * C)),
                  full((HEADS, C)), full((HEADS, C)), full((1, C)),
                  full((1, C)), full((1, C)), full((C, HEADS * C)),
                  full((HEADS, C)), full((HEADS, C)), full((1, C)),
                  full((1, C)), full((1, C))],
        out_specs=full((B, C)),
        out_shape=jax.ShapeDtypeStruct((B, C), jnp.float32),
        scratch_shapes=[pltpu.VMEM((grid, 3, CB), jnp.float32)],
    )(x, r, d, tok, W0, as0, ad0, b0, g0, be0, W1, as1, ad1, b1, g1, be1)


def _adj_mask():
    r = lax.broadcasted_iota(jnp.int32, (N, N), 0)
    c = lax.broadcasted_iota(jnp.int32, (N, N), 1)
    same = (r // 4) == (c // 4)
    nr, nc = r % 4, c % 4
    adj = (nc >= 1) | ((nr == 0) & (nc == 0))
    return same & adj


def _gat_layer(g, Wm, a_s, a_d, bb, mask, maskf):
    h = jnp.dot(g, Wm, preferred_element_type=jnp.float32)
    acc = jnp.zeros((N, C), jnp.float32)
    for hd in range(HEADS):
        hh = h[:, hd * C:(hd + 1) * C]
        a_s_h = a_s[hd:hd + 1, :]
        a_d_h = a_d[hd:hd + 1, :]
        al_s = lax.dot_general(a_s_h, hh, (((1,), (1,)), ((), ())),
                               preferred_element_type=jnp.float32)
        al_d = lax.dot_general(hh, a_d_h, (((1,), (1,)), ((), ())),
                               preferred_element_type=jnp.float32)
        e = al_d + al_s
        e = jnp.where(e > 0, e, 0.2 * e)
        e = jnp.where(mask, e, -1e30)
        m = jnp.max(e, axis=1, keepdims=True)
        ex = jnp.exp(e - m) * maskf
        ssum = jnp.sum(ex, axis=1, keepdims=True) + 1e-16
        alpha = ex / ssum
        acc = acc + jnp.dot(alpha, hh, preferred_element_type=jnp.float32)
    return acc * (1.0 / HEADS) + bb


def _ln(x, g, b):
    mu = jnp.mean(x, axis=-1, keepdims=True)
    xc = x - mu
    var = jnp.mean(xc * xc, axis=-1, keepdims=True)
    return xc * lax.rsqrt(var + 1e-5) * g + b


def _gnn_compute(feats, W0_ref, as0_ref, ad0_ref, b0_ref,
                 g0_ref, be0_ref, W1_ref, as1_ref, ad1_ref, b1_ref, g1_ref,
                 be1_ref, o_ref):
    mask = _adj_mask()
    maskf = mask.astype(jnp.float32)

    g = feats
    for (Wr, ar_s, ar_d, br, lgr, lbr) in (
            (W0_ref, as0_ref, ad0_ref, b0_ref, g0_ref, be0_ref),
            (W1_ref, as1_ref, ad1_ref, b1_ref, g1_ref, be1_ref)):
        g = _gat_layer(g, Wr[...], ar_s[...], ar_d[...], br[...], mask,
                       maskf) + g
        g = _ln(g, lgr[...], lbr[...])
        g = jnp.maximum(g, 0.0)

    rr = lax.broadcasted_iota(jnp.int32, (B, N), 0)
    cc = lax.broadcasted_iota(jnp.int32, (B, N), 1)
    sel = (cc == rr * 4).astype(jnp.float32)
    gtok = jnp.dot(sel, g, preferred_element_type=jnp.float32)
    o_ref[...] = 1.0 + jax.nn.sigmoid(gtok)


def _scale_body(x_ref, s_ref, o_ref):
    i = pl.program_id(0)
    b = i // NC2
    c0 = (i % NC2) * CB2
    for k in range(CB2):
        o_ref[0, k] = x_ref[0, k] * s_ref[b, c0 + k]


def _scale(x, s):
    grid = ROWS // CB2
    bs = pl.BlockSpec((1, CB2, H, W), lambda i: (i // NC2, i % NC2, 0, 0))
    return pl.pallas_call(
        _scale_body,
        grid=(grid,),
        in_specs=[bs, pl.BlockSpec(memory_space=pltpu.SMEM)],
        out_specs=bs,
        out_shape=jax.ShapeDtypeStruct((B, C, H, W), jnp.float32),
        compiler_params=pltpu.CompilerParams(vmem_limit_bytes=63 * 2**20),
    )(x, s)


def kernel(x_ful, rgb, dep, tok, W0, a_src0, a_dst0, b0, g0, be0,
           W1, a_src1, a_dst1, b1, g1, be1):
    scale = _means_gnn(
        x_ful, rgb, dep, tok,
        W0, a_src0.reshape(HEADS, C), a_dst0.reshape(HEADS, C),
        b0.reshape(1, C), g0.reshape(1, C), be0.reshape(1, C),
        W1, a_src1.reshape(HEADS, C), a_dst1.reshape(HEADS, C),
        b1.reshape(1, C), g1.reshape(1, C), be1.reshape(1, C))

    return _scale(x_ful, scale)


# speedup vs baseline: 1.1153x; 1.0038x over previous
import jax
import jax.numpy as jnp
from jax import lax
from jax.experimental import pallas as pl
from jax.experimental.pallas import tpu as pltpu

B, C, H, W = 4, 192, 224, 224
HEADS = 4
N = B * 4
ROWS = B * C
CB = 32
NC = C // CB
CB2 = 64
NC2 = C // CB2


def _means_gnn_body(x_ref, r_ref, d_ref, tok_ref, W0_ref, as0_ref, ad0_ref,
                    b0_ref, g0_ref, be0_ref, W1_ref, as1_ref, ad1_ref,
                    b1_ref, g1_ref, be1_ref, o_ref, acc_ref):
    grid = ROWS // CB
    i = pl.program_id(0)
    inv = 1.0 / (H * W)
    acc_ref[pl.ds(i, 1), 0:1, :] = jnp.sum(
        x_ref[...], axis=(2, 3)).reshape(1, 1, CB) * inv
    acc_ref[pl.ds(i, 1), 1:2, :] = jnp.sum(
        r_ref[...], axis=(2, 3)).reshape(1, 1, CB) * inv
    acc_ref[pl.ds(i, 1), 2:3, :] = jnp.sum(
        d_ref[...], axis=(2, 3)).reshape(1, 1, CB) * inv

    @pl.when(i == grid - 1)
    def _gnn_step():
        rows = []
        for b in range(B):
            rows.append(tok_ref[...])
            for t in range(3):
                rows.append(jnp.concatenate(
                    [acc_ref[b * NC + j, t:t + 1, :] for j in range(NC)],
                    axis=1))
        feats = jnp.concatenate(rows, axis=0)
        _gnn_compute(feats, W0_ref, as0_ref, ad0_ref, b0_ref,
                     g0_ref, be0_ref, W1_ref, as1_ref, ad1_ref, b1_ref,
                     g1_ref, be1_ref, o_ref)


def _means_gnn(x, r, d, tok, W0, as0, ad0, b0, g0, be0, W1, as1, ad1, b1,
               g1, be1):
    grid = ROWS // CB
    bs = pl.BlockSpec((1, CB, H, W), lambda i: (i // NC, i % NC, 0, 0))
    full = lambda s: pl.BlockSpec(s, lambda i: (0,) * len(s))
    return pl.pallas_call(
        _means_gnn_body,
        grid=(grid,),
        in_specs=[bs, bs, bs,
                  full((1, C)), full((C, HEADS * C)),
                  full((HEADS, C)), full((HEADS, C)), full((1, C)),
                  full((1, C)), full((1, C)), full((C, HEADS * C)),
                  full((HEADS, C)), full((HEADS, C)), full((1, C)),
                  full((1, C)), full((1, C))],
        out_specs=full((B, C)),
        out_shape=jax.ShapeDtypeStruct((B, C), jnp.float32),
        scratch_shapes=[pltpu.VMEM((grid, 3, CB), jnp.float32)],
    )(x, r, d, tok, W0, as0, ad0, b0, g0, be0, W1, as1, ad1, b1, g1, be1)


def _adj_mask():
    r = lax.broadcasted_iota(jnp.int32, (N, N), 0)
    c = lax.broadcasted_iota(jnp.int32, (N, N), 1)
    same = (r // 4) == (c // 4)
    nr, nc = r % 4, c % 4
    adj = (nc >= 1) | ((nr == 0) & (nc == 0))
    return same & adj


def _gat_layer(g, Wm, a_s, a_d, bb, mask, maskf):
    h = jnp.dot(g, Wm, preferred_element_type=jnp.float32)
    acc = jnp.zeros((N, C), jnp.float32)
    for hd in range(HEADS):
        hh = h[:, hd * C:(hd + 1) * C]
        a_s_h = a_s[hd:hd + 1, :]
        a_d_h = a_d[hd:hd + 1, :]
        al_s = lax.dot_general(a_s_h, hh, (((1,), (1,)), ((), ())),
                               preferred_element_type=jnp.float32)
        al_d = lax.dot_general(hh, a_d_h, (((1,), (1,)), ((), ())),
                               preferred_element_type=jnp.float32)
        e = al_d + al_s
        e = jnp.where(e > 0, e, 0.2 * e)
        e = jnp.where(mask, e, -1e30)
        m = jnp.max(e, axis=1, keepdims=True)
        ex = jnp.exp(e - m) * maskf
        ssum = jnp.sum(ex, axis=1, keepdims=True) + 1e-16
        alpha = ex / ssum
        acc = acc + jnp.dot(alpha, hh, preferred_element_type=jnp.float32)
    return acc * (1.0 / HEADS) + bb


def _ln(x, g, b):
    mu = jnp.mean(x, axis=-1, keepdims=True)
    xc = x - mu
    var = jnp.mean(xc * xc, axis=-1, keepdims=True)
    return xc * lax.rsqrt(var + 1e-5) * g + b


def _gnn_compute(feats, W0_ref, as0_ref, ad0_ref, b0_ref,
                 g0_ref, be0_ref, W1_ref, as1_ref, ad1_ref, b1_ref, g1_ref,
                 be1_ref, o_ref):
    mask = _adj_mask()
    maskf = mask.astype(jnp.float32)

    g = feats
    for (Wr, ar_s, ar_d, br, lgr, lbr) in (
            (W0_ref, as0_ref, ad0_ref, b0_ref, g0_ref, be0_ref),
            (W1_ref, as1_ref, ad1_ref, b1_ref, g1_ref, be1_ref)):
        g = _gat_layer(g, Wr[...], ar_s[...], ar_d[...], br[...], mask,
                       maskf) + g
        g = _ln(g, lgr[...], lbr[...])
        g = jnp.maximum(g, 0.0)

    rr = lax.broadcasted_iota(jnp.int32, (B, N), 0)
    cc = lax.broadcasted_iota(jnp.int32, (B, N), 1)
    sel = (cc == rr * 4).astype(jnp.float32)
    gtok = jnp.dot(sel, g, preferred_element_type=jnp.float32)
    o_ref[...] = 1.0 + jax.nn.sigmoid(gtok)


def _scale_body(x_ref, s_ref, o_ref):
    i = pl.program_id(0)
    b = i // NC2
    c0 = (i % NC2) * CB2
    for k in range(CB2):
        o_ref[0, k] = x_ref[0, k] * s_ref[b, c0 + k]


def _scale(x, s):
    grid = ROWS // CB2
    bs = pl.BlockSpec((1, CB2, H, W), lambda i: (i // NC2, i % NC2, 0, 0))
    return pl.pallas_call(
        _scale_body,
        grid=(grid,),
        in_specs=[bs, pl.BlockSpec(memory_space=pltpu.SMEM)],
        out_specs=bs,
        out_shape=jax.ShapeDtypeStruct((B, C, H, W), jnp.float32),
        compiler_params=pltpu.CompilerParams(vmem_limit_bytes=63 * 2**20),
    )(x, s)


def kernel(x_ful, rgb, dep, tok, W0, a_src0, a_dst0, b0, g0, be0,
           W1, a_src1, a_dst1, b1, g1, be1):
    scale = _means_gnn(
        x_ful, rgb, dep, tok,
        W0, a_src0.reshape(HEADS, C), a_dst0.reshape(HEADS, C),
        b0.reshape(1, C), g0.reshape(1, C), be0.reshape(1, C),
        W1, a_src1.reshape(HEADS, C), a_dst1.reshape(HEADS, C),
        b1.reshape(1, C), g1.reshape(1, C), be1.reshape(1, C))

    return _scale(x_ful, scale)
